# baseline (speedup 1.0000x reference)
import functools
import math

import jax
import jax.numpy as jnp
from jax import lax
from jax.experimental import pallas as pl
from jax.experimental.pallas import tpu as pltpu
from jax.experimental.pallas import tpu_sc as plsc

D_MODEL = 768
NUM_EXPERTS = 8
D_FF = 3072
SEQ = 2048
CAPACITY = 512
LANES = 16
NUM_SLOTS = NUM_EXPERTS * CAPACITY

BLK_F = 1536
BLK_R = 128
N_FB = D_FF // BLK_F

_MESH = plsc.VectorSubcoreMesh(core_axis_name="c", subcore_axis_name="s")
_SC_PARAMS = pltpu.CompilerParams(needs_layout_passes=False)


def _worker_id():
    return lax.axis_index("s") * 2 + lax.axis_index("c")


def _gate_body(gw_ref, tok_ref, out_ref):
    out_ref[...] = lax.dot_general(
        gw_ref[...], tok_ref[...],
        dimension_numbers=(((1,), (1,)), ((), ())),
        preferred_element_type=jnp.float32,
    )


def _gate(tokens, gate_w):
    return pl.pallas_call(
        _gate_body,
        out_shape=jax.ShapeDtypeStruct((NUM_EXPERTS, SEQ), jnp.float32),
    )(gate_w, tokens)


@functools.partial(
    pl.kernel,
    out_type=[
        jax.ShapeDtypeStruct((NUM_SLOTS,), jnp.int32),
        jax.ShapeDtypeStruct((NUM_SLOTS,), jnp.float32),
        jax.ShapeDtypeStruct((LANES,), jnp.int32),
    ],
    mesh=_MESH,
    compiler_params=_SC_PARAMS,
    scratch_types=[
        pltpu.VMEM((NUM_EXPERTS, SEQ), jnp.float32),
        pltpu.VMEM((NUM_SLOTS,), jnp.int32),
        pltpu.VMEM((NUM_SLOTS,), jnp.float32),
        pltpu.VMEM((LANES,), jnp.int32),
    ],
)
def _route(lgt_hbm, stt_hbm, ss_hbm, cnt_hbm, lg_v, stt_v, ss_v, cnt_v):
    wid = _worker_id()

    @pl.when(wid == 0)
    def _():
        pltpu.sync_copy(lgt_hbm, lg_v)

        def zero_body(i, _):
            base = i * LANES
            stt_v[pl.ds(base, LANES)] = (
                base + lax.iota(jnp.int32, LANES)
            ) & (SEQ - 1)
            return 0
        lax.fori_loop(0, NUM_SLOTS // LANES, zero_body, 0)

        def body(v, counts):
            ls = [lg_v[e, pl.ds(v * LANES, LANES)] for e in range(NUM_EXPERTS)]
            m = ls[0]
            for e in range(1, NUM_EXPERTS):
                m = jnp.maximum(m, ls[e])
            eid = jnp.full((LANES,), NUM_EXPERTS - 1, jnp.int32)
            for e in range(NUM_EXPERTS - 2, -1, -1):
                eid = jnp.where(ls[e] == m, e, eid)
            den = jnp.zeros((LANES,), jnp.float32)
            for e in range(NUM_EXPERTS):
                den = den + jnp.exp(ls[e] - m)
            prob = 1.0 / den

            rank = jnp.zeros((LANES,), jnp.int32)
            new_counts = []
            for e in range(NUM_EXPERTS):
                me = eid == e
                mi = jnp.where(me, 1, 0).astype(jnp.int32)
                cs = plsc.cumsum(mi)
                rank = jnp.where(me, cs - 1 + counts[e], rank)
                new_counts.append(counts[e] + jnp.sum(mi))

            kept = rank < CAPACITY
            tok = v * LANES + lax.iota(jnp.int32, LANES)
            slot = jnp.where(kept, eid * CAPACITY + rank, 0)
            plsc.store_scatter(stt_v, [slot], tok, mask=kept)
            plsc.store_scatter(ss_v, [slot], prob, mask=kept)
            return tuple(new_counts)

        counts = lax.fori_loop(
            0, SEQ // LANES, body, (jnp.int32(0),) * NUM_EXPERTS
        )

        cv = jnp.zeros((LANES,), jnp.int32)
        lane = lax.iota(jnp.int32, LANES)
        for e in range(NUM_EXPERTS):
            cv = jnp.where(lane == e, jnp.minimum(counts[e], CAPACITY), cv)
        cnt_v[...] = cv

        pltpu.sync_copy(stt_v, stt_hbm)
        pltpu.sync_copy(ss_v, ss_hbm)
        pltpu.sync_copy(cnt_v, cnt_hbm)


def _moe_body(stt_s, cnt_s, ss_s, tok_ref, w1_ref, w2_ref, out_ref,
              xb_ref, yb_ref):
    e = pl.program_id(0)
    fb = pl.program_id(1)
    cnt = cnt_s[e]

    @pl.when((e == 0) & (fb == 0))
    def _():
        out_ref[...] = jnp.zeros_like(out_ref)

    @pl.when(fb == 0)
    def _():
        def g(i, _):
            row = stt_s[e * CAPACITY + i]
            xb_ref[pl.ds(i, 1), :] = tok_ref[pl.ds(row, 1), :]
            return 0
        lax.fori_loop(0, cnt, g, 0)

    w1b = w1_ref[0].astype(jnp.bfloat16)
    w2b = w2_ref[0].astype(jnp.bfloat16)
    nblk = (cnt + BLK_R - 1) // BLK_R

    def body(rb, _):
        r0 = pl.multiple_of(rb * BLK_R, BLK_R)
        x = xb_ref[pl.ds(r0, BLK_R), :].astype(jnp.bfloat16)
        h = jax.nn.gelu(
            jnp.dot(x, w1b, preferred_element_type=jnp.float32)
        )
        part = jnp.dot(
            h.astype(jnp.bfloat16), w2b, preferred_element_type=jnp.float32
        )
        if N_FB == 1:
            yb_ref[pl.ds(r0, BLK_R), :] = part
        else:
            @pl.when(fb == 0)
            def _():
                yb_ref[pl.ds(r0, BLK_R), :] = part

            @pl.when(fb > 0)
            def _():
                yb_ref[pl.ds(r0, BLK_R), :] += part
        return 0

    lax.fori_loop(0, nblk, body, 0)

    @pl.when(fb == N_FB - 1)
    def _():
        def s(i, _):
            row = stt_s[e * CAPACITY + i]
            sc = ss_s[e * CAPACITY + i]
            out_ref[pl.ds(row, 1), :] = yb_ref[pl.ds(i, 1), :] * sc
            return 0
        lax.fori_loop(0, cnt, s, 0)


def _moe_ffn(stt, counts, sscale, tokens, w1, w2):
    grid_spec = pltpu.PrefetchScalarGridSpec(
        num_scalar_prefetch=3,
        grid=(NUM_EXPERTS, N_FB),
        in_specs=[
            pl.BlockSpec((SEQ, D_MODEL), lambda e, fb, *_: (0, 0)),
            pl.BlockSpec((1, D_MODEL, BLK_F), lambda e, fb, *_: (e, 0, fb)),
            pl.BlockSpec((1, BLK_F, D_MODEL), lambda e, fb, *_: (e, fb, 0)),
        ],
        out_specs=pl.BlockSpec((SEQ, D_MODEL), lambda e, fb, *_: (0, 0)),
        scratch_shapes=[
            pltpu.VMEM((CAPACITY, D_MODEL), jnp.float32),
            pltpu.VMEM((CAPACITY, D_MODEL), jnp.float32),
        ],
    )
    return pl.pallas_call(
        _moe_body,
        grid_spec=grid_spec,
        out_shape=jax.ShapeDtypeStruct((SEQ, D_MODEL), jnp.float32),
        compiler_params=pltpu.CompilerParams(
            dimension_semantics=("arbitrary", "arbitrary"),
        ),
    )(stt, counts, sscale, tokens, w1, w2)


def kernel(inputs, gate_w, w1, w2):
    tokens = inputs.reshape(-1, D_MODEL)
    logits_t = _gate(tokens, gate_w)
    stt, sscale, counts = _route(logits_t)
    out = _moe_ffn(stt, counts, sscale, tokens, w1, w2)
    return out.reshape(inputs.shape)


# speedup vs baseline: 1.0973x; 1.0973x over previous
import functools
import math

import jax
import jax.numpy as jnp
from jax import lax
from jax.experimental import pallas as pl
from jax.experimental.pallas import tpu as pltpu
from jax.experimental.pallas import tpu_sc as plsc

D_MODEL = 768
NUM_EXPERTS = 8
D_FF = 3072
SEQ = 2048
CAPACITY = 512
LANES = 16
NUM_SLOTS = NUM_EXPERTS * CAPACITY
NUM_WORKERS = 32

BLK_F = 1536
BLK_R = 128

_MESH = plsc.VectorSubcoreMesh(core_axis_name="c", subcore_axis_name="s")
_SC_PARAMS = pltpu.CompilerParams(needs_layout_passes=False)

_ROWS_PER_W = NUM_SLOTS // NUM_WORKERS
_TOKS_PER_W = SEQ // NUM_WORKERS


def _gate_body(gw_ref, tok_ref, out_ref):
    out_ref[...] = lax.dot_general(
        gw_ref[...], tok_ref[...],
        dimension_numbers=(((1,), (1,)), ((), ())),
        preferred_element_type=jnp.float32,
    )


def _gate(tokens, gate_w):
    return pl.pallas_call(
        _gate_body,
        out_shape=jax.ShapeDtypeStruct((NUM_EXPERTS, SEQ), jnp.float32),
    )(gate_w, tokens)


@functools.partial(
    pl.kernel,
    out_type=[
        jax.ShapeDtypeStruct((NUM_SLOTS, D_MODEL), jnp.float32),
        jax.ShapeDtypeStruct((SEQ,), jnp.int32),
        jax.ShapeDtypeStruct((SEQ,), jnp.float32),
        jax.ShapeDtypeStruct((LANES,), jnp.int32),
    ],
    mesh=_MESH,
    compiler_params=_SC_PARAMS,
    scratch_types=[
        pltpu.VMEM((NUM_EXPERTS, SEQ // LANES), jnp.float32),
        pltpu.VMEM((SEQ // LANES,), jnp.int32),
        pltpu.VMEM((SEQ // LANES,), jnp.float32),
        pltpu.VMEM((SEQ,), jnp.int32),
        pltpu.VMEM((SEQ,), jnp.float32),
        pltpu.VMEM((NUM_SLOTS,), jnp.int32),
        pltpu.VMEM((SEQ,), jnp.int32),
        pltpu.VMEM((SEQ,), jnp.float32),
        pltpu.VMEM((LANES,), jnp.int32),
        pltpu.VMEM_SHARED((NUM_SLOTS,), jnp.int32),
        pltpu.VMEM_SHARED((SEQ,), jnp.int32),
        pltpu.VMEM_SHARED((SEQ,), jnp.float32),
        pltpu.VMEM((_ROWS_PER_W,), jnp.int32),
        pltpu.VMEM((_ROWS_PER_W, D_MODEL), jnp.float32),
        pltpu.SemaphoreType.DMA,
    ],
)
def _route_dispatch(lgt_hbm, tok_hbm, dsp_hbm, tts_hbm, scale_hbm, cnt_hbm,
                    lgl_v, eidl_v, probl_v, eid_v, prob_v,
                    stt_v, tts_v, scale_v, cnt_v, stt_sh, eid_sh, prob_sh,
                    idx_v, rows_v, sem):
    c = lax.axis_index("c")
    s = lax.axis_index("s")
    wid = s * 2 + c
    tpw = SEQ // LANES

    t0 = s * tpw
    for e in range(NUM_EXPERTS):
        pltpu.sync_copy(lgt_hbm.at[e, pl.ds(t0, tpw)], lgl_v.at[e])

    def p1_body(v, _):
        ls = [lgl_v[e, pl.ds(v * LANES, LANES)] for e in range(NUM_EXPERTS)]
        m = ls[0]
        for e in range(1, NUM_EXPERTS):
            m = jnp.maximum(m, ls[e])
        eid = jnp.full((LANES,), NUM_EXPERTS - 1, jnp.int32)
        for e in range(NUM_EXPERTS - 2, -1, -1):
            eid = jnp.where(ls[e] == m, e, eid)
        den = jnp.zeros((LANES,), jnp.float32)
        for e in range(NUM_EXPERTS):
            den = den + jnp.exp(ls[e] - m)
        eidl_v[pl.ds(v * LANES, LANES)] = eid
        probl_v[pl.ds(v * LANES, LANES)] = 1.0 / den
        return 0
    lax.fori_loop(0, tpw // LANES, p1_body, 0)
    pltpu.sync_copy(eidl_v, eid_sh.at[pl.ds(t0, tpw)])
    pltpu.sync_copy(probl_v, prob_sh.at[pl.ds(t0, tpw)])

    plsc.subcore_barrier()

    @pl.when(s == 0)
    def _():
        pltpu.sync_copy(eid_sh, eid_v)
        pltpu.sync_copy(prob_sh, prob_v)

        def zero_body(i, _):
            base = i * LANES
            stt_v[pl.ds(base, LANES)] = (
                base + lax.iota(jnp.int32, LANES)
            ) & (SEQ - 1)
            return 0
        lax.fori_loop(0, NUM_SLOTS // LANES, zero_body, 0)

        def body(v, counts):
            eid = eid_v[pl.ds(v * LANES, LANES)]
            prob = prob_v[pl.ds(v * LANES, LANES)]
            rank = jnp.zeros((LANES,), jnp.int32)
            new_counts = []
            for e in range(NUM_EXPERTS):
                me = eid == e
                mi = jnp.where(me, 1, 0).astype(jnp.int32)
                cs = plsc.cumsum(mi)
                rank = jnp.where(me, cs - 1 + counts[e], rank)
                new_counts.append(counts[e] + jnp.sum(mi))

            kept = rank < CAPACITY
            tok = v * LANES + lax.iota(jnp.int32, LANES)
            slot = jnp.where(kept, eid * CAPACITY + rank, 0)
            tts_v[pl.ds(v * LANES, LANES)] = slot
            scale_v[pl.ds(v * LANES, LANES)] = jnp.where(kept, prob, 0.0)
            plsc.store_scatter(stt_v, [slot], tok, mask=kept)
            return tuple(new_counts)

        counts = lax.fori_loop(
            0, SEQ // LANES, body, (jnp.int32(0),) * NUM_EXPERTS
        )

        pltpu.sync_copy(stt_v, stt_sh)

        @pl.when(c == 0)
        def _():
            cv = jnp.zeros((LANES,), jnp.int32)
            lane = lax.iota(jnp.int32, LANES)
            for e in range(NUM_EXPERTS):
                cv = jnp.where(lane == e, jnp.minimum(counts[e], CAPACITY), cv)
            cnt_v[...] = cv
            pltpu.sync_copy(tts_v, tts_hbm)
            pltpu.sync_copy(scale_v, scale_hbm)
            pltpu.sync_copy(cnt_v, cnt_hbm)

    plsc.subcore_barrier()

    base = wid * _ROWS_PER_W
    pltpu.sync_copy(stt_sh.at[pl.ds(base, _ROWS_PER_W)], idx_v)
    pltpu.async_copy(tok_hbm.at[idx_v], rows_v, sem).wait()
    pltpu.sync_copy(rows_v, dsp_hbm.at[pl.ds(base, _ROWS_PER_W)])


def _ffn_body(cnt_ref, x_ref, w1_ref, w2_ref, out_ref):
    e = pl.program_id(0)
    fb = pl.program_id(1)
    nblk = (cnt_ref[e] + BLK_R - 1) // BLK_R

    @pl.when(fb == 0)
    def _():
        out_ref[...] = jnp.zeros_like(out_ref)

    w1b = w1_ref[0].astype(jnp.bfloat16)
    w2b = w2_ref[0].astype(jnp.bfloat16)

    def body(rb, _):
        r0 = pl.multiple_of(rb * BLK_R, BLK_R)
        x = x_ref[pl.ds(r0, BLK_R), :].astype(jnp.bfloat16)
        h = jax.nn.gelu(
            jnp.dot(x, w1b, preferred_element_type=jnp.float32)
        )
        out_ref[pl.ds(r0, BLK_R), :] += jnp.dot(
            h.astype(jnp.bfloat16), w2b, preferred_element_type=jnp.float32
        )
        return 0

    lax.fori_loop(0, nblk, body, 0)


def _ffn(counts, dispatch, w1, w2):
    grid_spec = pltpu.PrefetchScalarGridSpec(
        num_scalar_prefetch=1,
        grid=(NUM_EXPERTS, D_FF // BLK_F),
        in_specs=[
            pl.BlockSpec((CAPACITY, D_MODEL), lambda e, fb, *_: (e, 0)),
            pl.BlockSpec((1, D_MODEL, BLK_F), lambda e, fb, *_: (e, 0, fb)),
            pl.BlockSpec((1, BLK_F, D_MODEL), lambda e, fb, *_: (e, fb, 0)),
        ],
        out_specs=pl.BlockSpec((CAPACITY, D_MODEL), lambda e, fb, *_: (e, 0)),
    )
    return pl.pallas_call(
        _ffn_body,
        grid_spec=grid_spec,
        out_shape=jax.ShapeDtypeStruct((NUM_SLOTS, D_MODEL), jnp.float32),
        compiler_params=pltpu.CompilerParams(
            dimension_semantics=("arbitrary", "arbitrary"),
        ),
    )(counts, dispatch, w1, w2)


@functools.partial(
    pl.kernel,
    out_type=jax.ShapeDtypeStruct((SEQ, D_MODEL), jnp.float32),
    mesh=_MESH,
    compiler_params=_SC_PARAMS,
    scratch_types=[
        pltpu.VMEM((_TOKS_PER_W,), jnp.int32),
        pltpu.VMEM((_TOKS_PER_W,), jnp.float32),
        pltpu.VMEM((_TOKS_PER_W, D_MODEL), jnp.float32),
        pltpu.SemaphoreType.DMA,
    ],
)
def _combine(y_hbm, tts_hbm, scale_hbm, out_hbm, idx_v, sc_v, rows_v, sem):
    base = (lax.axis_index("s") * 2 + lax.axis_index("c")) * _TOKS_PER_W
    pltpu.sync_copy(tts_hbm.at[pl.ds(base, _TOKS_PER_W)], idx_v)
    pltpu.sync_copy(scale_hbm.at[pl.ds(base, _TOKS_PER_W)], sc_v)
    pltpu.async_copy(y_hbm.at[idx_v], rows_v, sem).wait()

    def body(i, _):
        sc = plsc.load_gather(sc_v, [jnp.zeros((LANES,), jnp.int32) + i])
        for j in range(D_MODEL // LANES):
            rows_v[i, pl.ds(j * LANES, LANES)] = (
                rows_v[i, pl.ds(j * LANES, LANES)] * sc
            )
        return 0

    lax.fori_loop(0, _TOKS_PER_W, body, 0)
    pltpu.sync_copy(rows_v, out_hbm.at[pl.ds(base, _TOKS_PER_W)])


def kernel(inputs, gate_w, w1, w2):
    tokens = inputs.reshape(-1, D_MODEL)
    logits_t = _gate(tokens, gate_w)
    dispatch, tts, scale, counts = _route_dispatch(logits_t, tokens)
    y = _ffn(counts, dispatch, w1, w2)
    out = _combine(y, tts, scale)
    return out.reshape(inputs.shape)


# speedup vs baseline: 1.1134x; 1.0147x over previous
import functools
import math

import jax
import jax.numpy as jnp
from jax import lax
from jax.experimental import pallas as pl
from jax.experimental.pallas import tpu as pltpu
from jax.experimental.pallas import tpu_sc as plsc

D_MODEL = 768
NUM_EXPERTS = 8
D_FF = 3072
SEQ = 2048
CAPACITY = 512
LANES = 16
NUM_SLOTS = NUM_EXPERTS * CAPACITY
NUM_WORKERS = 32

BLK_F = 1536
BLK_R = 128

_MESH = plsc.VectorSubcoreMesh(core_axis_name="c", subcore_axis_name="s")
_SC_PARAMS = pltpu.CompilerParams(needs_layout_passes=False)

_ROWS_PER_W = NUM_SLOTS // NUM_WORKERS
_TOKS_PER_W = SEQ // NUM_WORKERS


def _gate_body(gw_ref, tok_ref, out_ref):
    out_ref[...] = lax.dot_general(
        gw_ref[...], tok_ref[...],
        dimension_numbers=(((1,), (1,)), ((), ())),
        preferred_element_type=jnp.float32,
    )


def _gate(tokens, gate_w):
    return pl.pallas_call(
        _gate_body,
        out_shape=jax.ShapeDtypeStruct((NUM_EXPERTS, SEQ), jnp.float32),
    )(gate_w, tokens)


@functools.partial(
    pl.kernel,
    out_type=[
        jax.ShapeDtypeStruct((NUM_SLOTS, D_MODEL), jnp.float32),
        jax.ShapeDtypeStruct((SEQ,), jnp.int32),
        jax.ShapeDtypeStruct((SEQ,), jnp.float32),
        jax.ShapeDtypeStruct((LANES,), jnp.int32),
    ],
    mesh=_MESH,
    compiler_params=_SC_PARAMS,
    scratch_types=[
        pltpu.VMEM((NUM_EXPERTS, SEQ), jnp.float32),
        pltpu.VMEM((NUM_SLOTS,), jnp.int32),
        pltpu.VMEM((SEQ,), jnp.int32),
        pltpu.VMEM((SEQ,), jnp.float32),
        pltpu.VMEM((LANES,), jnp.int32),
        pltpu.VMEM_SHARED((NUM_SLOTS,), jnp.int32),
        pltpu.VMEM((_ROWS_PER_W,), jnp.int32),
        pltpu.VMEM((_ROWS_PER_W, D_MODEL), jnp.float32),
        pltpu.SemaphoreType.DMA,
    ],
)
def _route_dispatch(lgt_hbm, tok_hbm, dsp_hbm, tts_hbm, scale_hbm, cnt_hbm,
                    lg_v, stt_v, tts_v, scale_v, cnt_v, stt_sh,
                    idx_v, rows_v, sem):
    c = lax.axis_index("c")
    s = lax.axis_index("s")
    wid = s * 2 + c

    @pl.when(s == 0)
    def _():
        pltpu.sync_copy(lgt_hbm, lg_v)

        def zero_body(i, _):
            base = i * LANES
            stt_v[pl.ds(base, LANES)] = (
                base + lax.iota(jnp.int32, LANES)
            ) & (SEQ - 1)
            return 0
        lax.fori_loop(0, NUM_SLOTS // LANES, zero_body, 0)

        def body(v, counts):
            ls = [lg_v[e, pl.ds(v * LANES, LANES)] for e in range(NUM_EXPERTS)]
            m = ls[0]
            for e in range(1, NUM_EXPERTS):
                m = jnp.maximum(m, ls[e])
            eid = jnp.full((LANES,), NUM_EXPERTS - 1, jnp.int32)
            for e in range(NUM_EXPERTS - 2, -1, -1):
                eid = jnp.where(ls[e] == m, e, eid)
            den = jnp.zeros((LANES,), jnp.float32)
            for e in range(NUM_EXPERTS):
                den = den + jnp.exp(ls[e] - m)
            prob = 1.0 / den

            rank = jnp.zeros((LANES,), jnp.int32)
            new_counts = []
            for e in range(NUM_EXPERTS):
                me = eid == e
                mi = jnp.where(me, 1, 0).astype(jnp.int32)
                cs = plsc.cumsum(mi)
                rank = jnp.where(me, cs - 1 + counts[e], rank)
                new_counts.append(counts[e] + jnp.sum(mi))

            kept = rank < CAPACITY
            tok = v * LANES + lax.iota(jnp.int32, LANES)
            slot = jnp.where(kept, eid * CAPACITY + rank, 0)
            tts_v[pl.ds(v * LANES, LANES)] = slot
            scale_v[pl.ds(v * LANES, LANES)] = jnp.where(kept, prob, 0.0)
            plsc.store_scatter(stt_v, [slot], tok, mask=kept)
            return tuple(new_counts)

        counts = lax.fori_loop(
            0, SEQ // LANES, body, (jnp.int32(0),) * NUM_EXPERTS
        )

        pltpu.sync_copy(stt_v, stt_sh)

        @pl.when(c == 0)
        def _():
            cv = jnp.zeros((LANES,), jnp.int32)
            lane = lax.iota(jnp.int32, LANES)
            for e in range(NUM_EXPERTS):
                cv = jnp.where(lane == e, jnp.minimum(counts[e], CAPACITY), cv)
            cnt_v[...] = cv
            pltpu.sync_copy(tts_v, tts_hbm)
            pltpu.sync_copy(scale_v, scale_hbm)
            pltpu.sync_copy(cnt_v, cnt_hbm)

    plsc.subcore_barrier()

    base = wid * _ROWS_PER_W
    pltpu.sync_copy(stt_sh.at[pl.ds(base, _ROWS_PER_W)], idx_v)
    pltpu.async_copy(tok_hbm.at[idx_v], rows_v, sem).wait()
    pltpu.sync_copy(rows_v, dsp_hbm.at[pl.ds(base, _ROWS_PER_W)])


def _ffn_body(cnt_ref, x_ref, w1_ref, w2_ref, out_ref):
    e = pl.program_id(0)
    fb = pl.program_id(1)
    nblk = (cnt_ref[e] + BLK_R - 1) // BLK_R

    @pl.when(fb == 0)
    def _():
        out_ref[...] = jnp.zeros_like(out_ref)

    w1b = w1_ref[0].astype(jnp.bfloat16)
    w2b = w2_ref[0].astype(jnp.bfloat16)

    def body(rb, _):
        r0 = pl.multiple_of(rb * BLK_R, BLK_R)
        x = x_ref[pl.ds(r0, BLK_R), :].astype(jnp.bfloat16)
        h = jax.nn.gelu(
            jnp.dot(x, w1b, preferred_element_type=jnp.float32)
        )
        out_ref[pl.ds(r0, BLK_R), :] += jnp.dot(
            h.astype(jnp.bfloat16), w2b, preferred_element_type=jnp.float32
        )
        return 0

    lax.fori_loop(0, nblk, body, 0)


def _ffn(counts, dispatch, w1, w2):
    grid_spec = pltpu.PrefetchScalarGridSpec(
        num_scalar_prefetch=1,
        grid=(NUM_EXPERTS, D_FF // BLK_F),
        in_specs=[
            pl.BlockSpec((CAPACITY, D_MODEL), lambda e, fb, *_: (e, 0)),
            pl.BlockSpec((1, D_MODEL, BLK_F), lambda e, fb, *_: (e, 0, fb)),
            pl.BlockSpec((1, BLK_F, D_MODEL), lambda e, fb, *_: (e, fb, 0)),
        ],
        out_specs=pl.BlockSpec((CAPACITY, D_MODEL), lambda e, fb, *_: (e, 0)),
    )
    return pl.pallas_call(
        _ffn_body,
        grid_spec=grid_spec,
        out_shape=jax.ShapeDtypeStruct((NUM_SLOTS, D_MODEL), jnp.float32),
        compiler_params=pltpu.CompilerParams(
            dimension_semantics=("arbitrary", "arbitrary"),
        ),
    )(counts, dispatch, w1, w2)


@functools.partial(
    pl.kernel,
    out_type=jax.ShapeDtypeStruct((SEQ, D_MODEL), jnp.float32),
    mesh=_MESH,
    compiler_params=_SC_PARAMS,
    scratch_types=[
        pltpu.VMEM((_TOKS_PER_W,), jnp.int32),
        pltpu.VMEM((_TOKS_PER_W,), jnp.float32),
        pltpu.VMEM((_TOKS_PER_W, D_MODEL), jnp.float32),
        pltpu.SemaphoreType.DMA,
    ],
)
def _combine(y_hbm, tts_hbm, scale_hbm, out_hbm, idx_v, sc_v, rows_v, sem):
    base = (lax.axis_index("s") * 2 + lax.axis_index("c")) * _TOKS_PER_W
    pltpu.sync_copy(tts_hbm.at[pl.ds(base, _TOKS_PER_W)], idx_v)
    pltpu.sync_copy(scale_hbm.at[pl.ds(base, _TOKS_PER_W)], sc_v)
    pltpu.async_copy(y_hbm.at[idx_v], rows_v, sem).wait()

    def body(i, _):
        sc = plsc.load_gather(sc_v, [jnp.zeros((LANES,), jnp.int32) + i])
        for j in range(D_MODEL // LANES):
            rows_v[i, pl.ds(j * LANES, LANES)] = (
                rows_v[i, pl.ds(j * LANES, LANES)] * sc
            )
        return 0

    lax.fori_loop(0, _TOKS_PER_W, body, 0)
    pltpu.sync_copy(rows_v, out_hbm.at[pl.ds(base, _TOKS_PER_W)])


def kernel(inputs, gate_w, w1, w2):
    tokens = inputs.reshape(-1, D_MODEL)
    logits_t = _gate(tokens, gate_w)
    dispatch, tts, scale, counts = _route_dispatch(logits_t, tokens)
    y = _ffn(counts, dispatch, w1, w2)
    out = _combine(y, tts, scale)
    return out.reshape(inputs.shape)
